# baseline (speedup 1.0000x reference)
import functools

import jax
import jax.numpy as jnp
from jax import lax
from jax.experimental import pallas as pl
from jax.experimental.pallas import tpu as pltpu
from jax.experimental.pallas import tpu_sc as plsc

BATCH = 16384
FACTORS = 32

_INFO = plsc.get_sparse_core_info()
_NC = _INFO.num_cores
_NS = _INFO.num_subcores
_NW = _NC * _NS
_BPW = BATCH // _NW
_CHUNK = 128
_NCHUNK = _BPW // _CHUNK


def _sc_gather_pair(user_idx, item_idx, user_emb, item_emb):
    mesh = plsc.VectorSubcoreMesh(core_axis_name="c", subcore_axis_name="s")

    @functools.partial(
        pl.kernel,
        mesh=mesh,
        out_type=[
            jax.ShapeDtypeStruct((BATCH, FACTORS), jnp.float32),
            jax.ShapeDtypeStruct((BATCH, FACTORS), jnp.float32),
        ],
        scratch_types=[
            pltpu.VMEM((_BPW,), jnp.int32),
            pltpu.VMEM((_BPW,), jnp.int32),
            pltpu.VMEM((_BPW, FACTORS), jnp.float32),
            pltpu.VMEM((_BPW, FACTORS), jnp.float32),
            pltpu.SemaphoreType.DMA,
        ],
        compiler_params=pltpu.CompilerParams(use_tc_tiling_on_sc=False),
    )
    def k(uidx_hbm, iidx_hbm, uemb_hbm, iemb_hbm, u_out, v_out,
          uidx_v, iidx_v, urows_v, irows_v, sem):
        wid = lax.axis_index("s") * _NC + lax.axis_index("c")
        base = wid * _BPW
        pltpu.sync_copy(uidx_hbm.at[pl.ds(base, _BPW)], uidx_v)
        pltpu.sync_copy(iidx_hbm.at[pl.ds(base, _BPW)], iidx_v)
        copies = []
        for c in range(_NCHUNK):
            sl = pl.ds(c * _CHUNK, _CHUNK)
            copies.append(pltpu.async_copy(
                uemb_hbm.at[uidx_v.at[sl]], urows_v.at[sl], sem))
            copies.append(pltpu.async_copy(
                iemb_hbm.at[iidx_v.at[sl]], irows_v.at[sl], sem))
        for cp in copies:
            cp.wait()
        pltpu.sync_copy(urows_v, u_out.at[pl.ds(base, _BPW)])
        pltpu.sync_copy(irows_v, v_out.at[pl.ds(base, _BPW)])

    return k(user_idx, item_idx, user_emb, item_emb)


_BM = 2048


def _mlp_body(u_ref, v_ref, w1_ref, b1_ref, w2_ref, b2_ref, w3_ref, b3_ref,
              w4_ref, b4_ref, o_ref):
    f32 = jnp.float32
    w1 = w1_ref[...]
    h = (jnp.dot(u_ref[...], w1[:FACTORS], preferred_element_type=f32)
         + jnp.dot(v_ref[...], w1[FACTORS:], preferred_element_type=f32)
         + b1_ref[...])
    h = jnp.maximum(h, 0.0)
    h = jnp.dot(h, w2_ref[...], preferred_element_type=f32) + b2_ref[...]
    h = jnp.maximum(h, 0.0)
    h = jnp.dot(h, w3_ref[...], preferred_element_type=f32) + b3_ref[...]
    h = jnp.maximum(h, 0.0)
    s = jnp.sum(h * w4_ref[...], axis=1, keepdims=True) + b4_ref[...]
    o_ref[...] = jax.nn.sigmoid(s)


def _mlp(u, v, W1, b1, W2, b2, W3, b3, W4, b4):
    out = pl.pallas_call(
        _mlp_body,
        grid=(BATCH // _BM,),
        in_specs=[
            pl.BlockSpec((_BM, FACTORS), lambda i: (i, 0)),
            pl.BlockSpec((_BM, FACTORS), lambda i: (i, 0)),
            pl.BlockSpec((64, 64), lambda i: (0, 0)),
            pl.BlockSpec((1, 64), lambda i: (0, 0)),
            pl.BlockSpec((64, 32), lambda i: (0, 0)),
            pl.BlockSpec((1, 32), lambda i: (0, 0)),
            pl.BlockSpec((32, 16), lambda i: (0, 0)),
            pl.BlockSpec((1, 16), lambda i: (0, 0)),
            pl.BlockSpec((1, 16), lambda i: (0, 0)),
            pl.BlockSpec((1, 1), lambda i: (0, 0)),
        ],
        out_specs=pl.BlockSpec((_BM, 1), lambda i: (i, 0)),
        out_shape=jax.ShapeDtypeStruct((BATCH, 1), jnp.float32),
    )(u, v, W1, b1.reshape(1, 64), W2, b2.reshape(1, 32),
      W3, b3.reshape(1, 16), W4.reshape(1, 16), b4.reshape(1, 1))
    return jnp.squeeze(out, axis=-1)


def kernel(user_input, item_input, user_emb, item_emb,
           W1, b1, W2, b2, W3, b3, W4, b4):
    u, v = _sc_gather_pair(user_input, item_input, user_emb, item_emb)
    return _mlp(u, v, W1, b1, W2, b2, W3, b3, W4, b4)
